# Initial kernel scaffold; baseline (speedup 1.0000x reference)
import jax
import jax.numpy as jnp
from jax.experimental import pallas as pl


def kernel(x, codebook):
    raise NotImplementedError("write your pallas kernel here")


# speedup vs baseline: 3.8178x; 3.8178x over previous
import jax
import jax.numpy as jnp
from jax.experimental import pallas as pl

_HI = jax.lax.Precision.HIGHEST


def _vq_block_kernel(x_ref, cb_ref, quant_ref, idx_ref, loss_ref):
    x = x_ref[...]
    cb = cb_ref[...]
    bb = x.shape[0]
    k = cb.shape[0]

    cn = jnp.sum(cb * cb, axis=1, keepdims=True)
    cb_aug = jnp.concatenate([-2.0 * cb, cn], axis=1)
    x_aug = jnp.concatenate([x, jnp.ones((bb, 1), jnp.float32)], axis=1)
    d2 = jax.lax.dot_general(x_aug, cb_aug, (((1,), (1,)), ((), ())),
                             precision=_HI,
                             preferred_element_type=jnp.float32)

    iota = jax.lax.broadcasted_iota(jnp.int32, d2.shape, 1)
    m1 = jnp.min(d2, axis=1, keepdims=True)
    i1 = jnp.min(jnp.where(d2 == m1, iota, k), axis=1, keepdims=True)
    d2b = jnp.where(iota == i1, jnp.inf, d2)
    m2 = jnp.min(d2b, axis=1, keepdims=True)
    i2 = jnp.min(jnp.where(d2b == m2, iota, k), axis=1, keepdims=True)

    oh1 = (iota == i1).astype(jnp.float32)
    oh2 = (iota == i2).astype(jnp.float32)
    c1 = jax.lax.dot_general(oh1, cb, (((1,), (0,)), ((), ())),
                             precision=_HI,
                             preferred_element_type=jnp.float32)
    c2 = jax.lax.dot_general(oh2, cb, (((1,), (0,)), ((), ())),
                             precision=_HI,
                             preferred_element_type=jnp.float32)

    r1 = x - c1
    r2 = x - c2
    e1 = jnp.sum(r1 * r1, axis=1, keepdims=True)
    e2 = jnp.sum(r2 * r2, axis=1, keepdims=True)
    f1 = jnp.sqrt(e1)
    f2 = jnp.sqrt(e2)
    pick1 = (f1 < f2) | ((f1 == f2) & (i1 < i2))

    quant = jnp.where(pick1, c1, c2)
    diff = quant - x
    quant_ref[...] = x + diff
    idx_ref[...] = jnp.where(pick1, i1, i2)[:, 0]

    @pl.when(pl.program_id(0) == 0)
    def _init():
        loss_ref[...] = jnp.zeros((1, 1), jnp.float32)

    loss_ref[...] += jnp.sum(diff * diff, keepdims=True)


def kernel(x, codebook):
    b, d = x.shape
    k = codebook.shape[0]
    bb = 512
    grid = b // bb

    quant, idx, loss_sum = pl.pallas_call(
        _vq_block_kernel,
        grid=(grid,),
        in_specs=[
            pl.BlockSpec((bb, d), lambda i: (i, 0)),
            pl.BlockSpec((k, d), lambda i: (0, 0)),
        ],
        out_specs=[
            pl.BlockSpec((bb, d), lambda i: (i, 0)),
            pl.BlockSpec((bb,), lambda i: (i,)),
            pl.BlockSpec((1, 1), lambda i: (0, 0)),
        ],
        out_shape=[
            jax.ShapeDtypeStruct((b, d), jnp.float32),
            jax.ShapeDtypeStruct((b,), jnp.int32),
            jax.ShapeDtypeStruct((1, 1), jnp.float32),
        ],
    )(x, codebook)

    loss = loss_sum[0, 0] / jnp.float32(b * d)
    return (quant, loss, loss, idx)
